# Initial kernel scaffold; baseline (speedup 1.0000x reference)
import jax
import jax.numpy as jnp
from jax.experimental import pallas as pl


def kernel(encoder_hidden_states, labels, metadata, wte, wpe):
    raise NotImplementedError("write your pallas kernel here")


# speedup vs baseline: 1.0899x; 1.0899x over previous
import functools

import jax
import jax.numpy as jnp
from jax import lax
from jax.experimental import pallas as pl
from jax.experimental.pallas import tpu as pltpu
from jax.experimental.pallas import tpu_sc as plsc

PAD_ID = 0
START_ID = 0
LANES = 16


@functools.partial(jax.jit, static_argnames=("B", "S", "D"))
def _embed_lookup(flat_ids, wte, wpe, B, S, D):
    NC, NS = 2, 16
    NW = NC * NS
    CH = S // NW

    mesh = plsc.VectorSubcoreMesh(core_axis_name="c", subcore_axis_name="s")

    @functools.partial(
        pl.kernel,
        mesh=mesh,
        out_type=jax.ShapeDtypeStruct((B * S, D), jnp.float32),
        scratch_types=[
            pltpu.VMEM((CH,), jnp.int32),
            pltpu.VMEM((CH, D), jnp.float32),
            pltpu.VMEM((CH, D), jnp.float32),
            pltpu.SemaphoreType.DMA,
        ],
    )
    def k(ids_hbm, wte_hbm, wpe_hbm, out_hbm, idx_v, wpe_v, rows_v, sem):
        wid = lax.axis_index("s") * NC + lax.axis_index("c")
        s0 = wid * CH
        pltpu.sync_copy(wpe_hbm.at[pl.ds(s0, CH), :], wpe_v)

        def add_row(i, _):
            for j in range(D // LANES):
                sl = pl.ds(j * LANES, LANES)
                rows_v[i, sl] = rows_v[i, sl] + wpe_v[i, sl]
            return _

        for b in range(B):
            base = b * S + s0
            pltpu.sync_copy(ids_hbm.at[pl.ds(base, CH)], idx_v)
            pltpu.async_copy(wte_hbm.at[idx_v], rows_v, sem).wait()
            lax.fori_loop(0, CH, add_row, 0)
            pltpu.sync_copy(rows_v, out_hbm.at[pl.ds(base, CH), :])

    return k(flat_ids, wte, wpe)


def kernel(encoder_hidden_states, labels, metadata, wte, wpe):
    B, S = labels.shape
    D = wte.shape[1]

    ids = jnp.concatenate(
        [jnp.full((B, 1), START_ID, labels.dtype), labels[:, :-1]], axis=1
    )
    ids = jnp.where(ids == -100, PAD_ID, ids)

    token_emb = _embed_lookup(ids.reshape(-1), wte, wpe, B, S, D)
    token_emb = token_emb.reshape(B, S, D)

    enc_b, enc_s, _ = encoder_hidden_states.shape
    encoder_extended_attention_mask = jnp.zeros(
        (enc_b, 1, 1, enc_s), dtype=jnp.float32
    )

    return (
        encoder_hidden_states,
        token_emb,
        encoder_extended_attention_mask,
        metadata,
        ids,
        labels,
    )
